# baseline (speedup 1.0000x reference)
import functools

import jax
import jax.numpy as jnp
from jax import lax
from jax.experimental import pallas as pl
from jax.experimental.pallas import tpu as pltpu
from jax.experimental.pallas import tpu_sc as plsc

_INTERPRET = False

GS = 1024
E = 8
K = 2
CAP = 256
NW = 32
NSLOT = E * 4 * CAP
TRASH = NSLOT


def _router_kernel(x_ref, wr_ref,
                   dst_ref, r0_ref, r1_ref, c0_ref, c1_ref, valid_ref, aux_ref,
                   *, G):
    x = x_ref[...]
    wr = wr_ref[...]
    N = G * GS
    logits = jnp.dot(x, wr, preferred_element_type=jnp.float32)
    m = jnp.max(logits, axis=-1, keepdims=True)
    ex = jnp.exp(logits - m)
    gates = ex / jnp.sum(ex, axis=-1, keepdims=True)

    idx8 = lax.broadcasted_iota(jnp.int32, (N, E), 1)
    top1v = jnp.max(gates, axis=-1, keepdims=True)
    top1i = jnp.min(jnp.where(gates == top1v, idx8, E), axis=-1, keepdims=True)
    oh1 = idx8 == top1i
    masked = jnp.where(oh1, -1.0, gates)
    top2v = jnp.max(masked, axis=-1, keepdims=True)
    top2i = jnp.min(jnp.where(masked == top2v, idx8, E), axis=-1, keepdims=True)
    oh2 = idx8 == top2i

    M = jnp.concatenate([oh1.astype(jnp.float32).reshape(G, GS, E),
                         oh2.astype(jnp.float32).reshape(G, GS, E)], axis=1)
    C = M
    sh = 1
    while sh < K * GS:
        C = C + jnp.concatenate(
            [jnp.zeros((G, sh, E), jnp.float32), C[:, :-sh, :]], axis=1)
        sh *= 2
    P = C - 1.0
    pos_entry = jnp.sum(M * P, axis=-1)
    keepf = (pos_entry < CAP).astype(jnp.float32)
    keepb = pos_entry < CAP
    pos_i = jnp.minimum(pos_entry, CAP - 1).astype(jnp.int32)

    topi_flat = jnp.concatenate([top1i.reshape(G, GS), top2i.reshape(G, GS)],
                                axis=1)
    gi = lax.broadcasted_iota(jnp.int32, (G, K * GS), 0)
    slot = (topi_flat * G + gi) * CAP + pos_i
    dst = jnp.where(keepb, slot, TRASH)
    dst_ref[...] = dst.reshape(NW, K * GS * G // NW // 128, 128)

    r0 = jnp.where(keepb[:, :GS], slot[:, :GS], 0).reshape(N)
    r1 = jnp.where(keepb[:, GS:], slot[:, GS:], 0).reshape(N)
    r0_ref[...] = r0.reshape(NW, N // NW)
    r1_ref[...] = r1.reshape(NW, N // NW)
    c0_ref[...] = (top1v.reshape(G, GS) * keepf[:, :GS]).reshape(N, 1)
    c1_ref[...] = (top2v.reshape(G, GS) * keepf[:, GS:]).reshape(N, 1)

    cnt = jnp.minimum(C[:, K * GS - 1, :], float(CAP)).astype(jnp.int32)
    ci3 = lax.broadcasted_iota(jnp.int32, (G, E, CAP), 2)
    valid = (ci3 < cnt[:, :, None]).astype(jnp.float32)
    valid_ref[...] = valid.reshape(G, E, CAP, 1)

    imp = jnp.sum(gates.reshape(G, GS, E), axis=1)
    mu = jnp.mean(imp, axis=-1, keepdims=True)
    var = jnp.mean((imp - mu) ** 2, axis=-1, keepdims=True)
    aux_ref[...] = jnp.mean(var / (mu + 1e-10) ** 2).reshape(1, 1)


def _dispatch_sc(x_hbm, dst_hbm, xe_hbm, idx_v, rows_v, sem):
    wid = lax.axis_index("s") * 2 + lax.axis_index("c")
    g = wid // 8
    sub = wid % 8
    xbase = g * GS + (sub % 4) * 256
    pltpu.sync_copy(dst_hbm.at[wid], idx_v)
    for j in range(2):
        pltpu.sync_copy(x_hbm.at[pl.ds(xbase + j * 128, 128)], rows_v)
        pltpu.async_copy(rows_v, xe_hbm.at[idx_v.at[j]], sem).wait()


def _combine_sc(y_hbm, r0_hbm, r1_hbm, y0_hbm, y1_hbm, idx_v, buf_v, sem):
    wid = lax.axis_index("s") * 2 + lax.axis_index("c")
    base = wid * 128
    pltpu.sync_copy(r0_hbm.at[wid], idx_v)
    pltpu.async_copy(y_hbm.at[idx_v], buf_v, sem).wait()
    pltpu.sync_copy(buf_v, y0_hbm.at[pl.ds(base, 128)])
    pltpu.sync_copy(r1_hbm.at[wid], idx_v)
    pltpu.async_copy(y_hbm.at[idx_v], buf_v, sem).wait()
    pltpu.sync_copy(buf_v, y1_hbm.at[pl.ds(base, 128)])


def _mlp_kernel(xe_ref, valid_ref, w1_ref, b1_ref, w2_ref, b2_ref, y_ref):
    vm = valid_ref[0, 0]
    xe = jnp.where(vm > 0, xe_ref[0, 0], 0.0)
    h = jnp.dot(xe, w1_ref[0], preferred_element_type=jnp.float32) + b1_ref[0]
    h = jax.nn.gelu(h)
    y = jnp.dot(h, w2_ref[0], preferred_element_type=jnp.float32) + b2_ref[0]
    y_ref[0, 0] = y


def _wsum_kernel(c0_ref, c1_ref, y0_ref, y1_ref, out_ref):
    out_ref[...] = c0_ref[...] * y0_ref[...] + c1_ref[...] * y1_ref[...]


def kernel(inputs, w_router, w1, b1, w2, b2):
    b, s, d = inputs.shape
    G = (b * s) // GS
    N = G * GS
    MLP = w1.shape[2]
    x2 = inputs.reshape(N, d)

    dst, r0, r1, c0, c1, valid, aux = pl.pallas_call(
        functools.partial(_router_kernel, G=G),
        out_shape=[
            jax.ShapeDtypeStruct((NW, K * N // NW // 128, 128), jnp.int32),
            jax.ShapeDtypeStruct((NW, N // NW), jnp.int32),
            jax.ShapeDtypeStruct((NW, N // NW), jnp.int32),
            jax.ShapeDtypeStruct((N, 1), jnp.float32),
            jax.ShapeDtypeStruct((N, 1), jnp.float32),
            jax.ShapeDtypeStruct((G, E, CAP, 1), jnp.float32),
            jax.ShapeDtypeStruct((1, 1), jnp.float32),
        ],
        interpret=_INTERPRET,
    )(x2, w_router)

    mesh = plsc.VectorSubcoreMesh(core_axis_name="c", subcore_axis_name="s")

    xe_flat = pl.kernel(
        _dispatch_sc,
        mesh=mesh,
        out_type=jax.ShapeDtypeStruct((NSLOT + 8, d), jnp.float32),
        scratch_types=[
            pltpu.VMEM((2, 128), jnp.int32),
            pltpu.VMEM((128, d), jnp.float32),
            pltpu.SemaphoreType.DMA,
        ],
    )(x2, dst)

    xe = xe_flat[:NSLOT].reshape(E, G, CAP, d)
    validT = valid.transpose(1, 0, 2, 3)
    b1r = b1.reshape(E, 1, MLP)
    b2r = b2.reshape(E, 1, d)

    y = pl.pallas_call(
        _mlp_kernel,
        grid=(E, G),
        in_specs=[
            pl.BlockSpec((1, 1, CAP, d), lambda e, g: (e, g, 0, 0)),
            pl.BlockSpec((1, 1, CAP, 1), lambda e, g: (e, g, 0, 0)),
            pl.BlockSpec((1, d, MLP), lambda e, g: (e, 0, 0)),
            pl.BlockSpec((1, 1, MLP), lambda e, g: (e, 0, 0)),
            pl.BlockSpec((1, MLP, d), lambda e, g: (e, 0, 0)),
            pl.BlockSpec((1, 1, d), lambda e, g: (e, 0, 0)),
        ],
        out_specs=pl.BlockSpec((1, 1, CAP, d), lambda e, g: (e, g, 0, 0)),
        out_shape=jax.ShapeDtypeStruct((E, G, CAP, d), jnp.float32),
        interpret=_INTERPRET,
    )(xe, validT, w1, b1r, w2, b2r)

    y_flat = y.reshape(E * G * CAP, d)
    y0, y1 = pl.kernel(
        _combine_sc,
        mesh=mesh,
        out_type=[
            jax.ShapeDtypeStruct((N, d), jnp.float32),
            jax.ShapeDtypeStruct((N, d), jnp.float32),
        ],
        scratch_types=[
            pltpu.VMEM((128,), jnp.int32),
            pltpu.VMEM((128, d), jnp.float32),
            pltpu.SemaphoreType.DMA,
        ],
    )(y_flat, r0, r1)

    out2 = pl.pallas_call(
        _wsum_kernel,
        out_shape=jax.ShapeDtypeStruct((N, d), jnp.float32),
        interpret=_INTERPRET,
    )(c0, c1, y0, y1)

    out = out2.reshape(b, s, d)
    aux_s = aux[0, 0]
    return out, {"auxiliary_loss": aux_s, "importance_loss": aux_s}


# speedup vs baseline: 1.0560x; 1.0560x over previous
import functools

import jax
import jax.numpy as jnp
from jax import lax
from jax.experimental import pallas as pl
from jax.experimental.pallas import tpu as pltpu
from jax.experimental.pallas import tpu_sc as plsc

_INTERPRET = False

GS = 1024
E = 8
K = 2
CAP = 256
NW = 32
NSLOT = E * 4 * CAP
TRASH = NSLOT


def _router_kernel(x_ref, wr_ref,
                   dst_ref, r0_ref, r1_ref, c0_ref, c1_ref, valid_ref, aux_ref,
                   *, G):
    x = x_ref[...]
    wr = wr_ref[...]
    N = G * GS
    logits = jnp.dot(x, wr, preferred_element_type=jnp.float32)
    m = jnp.max(logits, axis=-1, keepdims=True)
    ex = jnp.exp(logits - m)
    gates = ex / jnp.sum(ex, axis=-1, keepdims=True)

    idx8 = lax.broadcasted_iota(jnp.int32, (N, E), 1)
    top1v = jnp.max(gates, axis=-1, keepdims=True)
    top1i = jnp.min(jnp.where(gates == top1v, idx8, E), axis=-1, keepdims=True)
    oh1 = idx8 == top1i
    masked = jnp.where(oh1, -1.0, gates)
    top2v = jnp.max(masked, axis=-1, keepdims=True)
    top2i = jnp.min(jnp.where(masked == top2v, idx8, E), axis=-1, keepdims=True)
    oh2 = idx8 == top2i

    M = jnp.concatenate([oh1.astype(jnp.float32).reshape(G, GS, E),
                         oh2.astype(jnp.float32).reshape(G, GS, E)], axis=1)
    C = M
    sh = 1
    while sh < K * GS:
        C = C + jnp.concatenate(
            [jnp.zeros((G, sh, E), jnp.float32), C[:, :-sh, :]], axis=1)
        sh *= 2
    P = C - 1.0
    pos_entry = jnp.sum(M * P, axis=-1)
    keepf = (pos_entry < CAP).astype(jnp.float32)
    keepb = pos_entry < CAP
    pos_i = jnp.minimum(pos_entry, CAP - 1).astype(jnp.int32)

    topi_flat = jnp.concatenate([top1i.reshape(G, GS), top2i.reshape(G, GS)],
                                axis=1)
    gi = lax.broadcasted_iota(jnp.int32, (G, K * GS), 0)
    slot = (topi_flat * G + gi) * CAP + pos_i
    dst = jnp.where(keepb, slot, TRASH)
    dst_ref[...] = dst.reshape(NW, K * GS * G // NW // 128, 128)

    r0 = jnp.where(keepb[:, :GS], slot[:, :GS], 0).reshape(N)
    r1 = jnp.where(keepb[:, GS:], slot[:, GS:], 0).reshape(N)
    r0_ref[...] = r0.reshape(NW, N // NW)
    r1_ref[...] = r1.reshape(NW, N // NW)
    cv0 = (top1v.reshape(G, GS) * keepf[:, :GS]).reshape(N, 1)
    cv1 = (top2v.reshape(G, GS) * keepf[:, GS:]).reshape(N, 1)
    c0_ref[...] = jnp.broadcast_to(cv0, (N, 16)).reshape(NW, N // NW, 16)
    c1_ref[...] = jnp.broadcast_to(cv1, (N, 16)).reshape(NW, N // NW, 16)

    cnt = jnp.minimum(C[:, K * GS - 1, :], float(CAP)).astype(jnp.int32)
    ci3 = lax.broadcasted_iota(jnp.int32, (G, E, CAP), 2)
    valid = (ci3 < cnt[:, :, None]).astype(jnp.float32)
    valid_ref[...] = valid.reshape(G, E, CAP, 1)

    imp = jnp.sum(gates.reshape(G, GS, E), axis=1)
    mu = jnp.mean(imp, axis=-1, keepdims=True)
    var = jnp.mean((imp - mu) ** 2, axis=-1, keepdims=True)
    aux_ref[...] = jnp.mean(var / (mu + 1e-10) ** 2).reshape(1, 1)


def _dispatch_sc(x_hbm, dst_hbm, xe_hbm, idx_v, rows_v, sem):
    wid = lax.axis_index("s") * 2 + lax.axis_index("c")
    g = wid // 8
    sub = wid % 8
    xbase = g * GS + (sub % 4) * 256
    pltpu.sync_copy(dst_hbm.at[wid], idx_v)
    for j in range(2):
        pltpu.sync_copy(x_hbm.at[pl.ds(xbase + j * 128, 128)], rows_v)
        pltpu.async_copy(rows_v, xe_hbm.at[idx_v.at[j]], sem).wait()


def _combine_sc(y_hbm, r0_hbm, r1_hbm, c0_hbm, c1_hbm, out_hbm,
                idx0_v, idx1_v, c0_v, c1_v, buf0_v, buf1_v, sem):
    wid = lax.axis_index("s") * 2 + lax.axis_index("c")
    tpw = 128
    half = 32
    d = buf0_v.shape[1]
    nv = d // 16
    pltpu.sync_copy(r0_hbm.at[wid], idx0_v)
    pltpu.sync_copy(r1_hbm.at[wid], idx1_v)
    pltpu.sync_copy(c0_hbm.at[wid], c0_v)
    pltpu.sync_copy(c1_hbm.at[wid], c1_v)
    for r in range(4):
        cp0 = pltpu.async_copy(y_hbm.at[idx0_v.at[pl.ds(r * half, half)]],
                               buf0_v, sem)
        cp1 = pltpu.async_copy(y_hbm.at[idx1_v.at[pl.ds(r * half, half)]],
                               buf1_v, sem)
        cp0.wait()
        cp1.wait()

        def body(t, _, r=r):
            ti = r * half + t
            g0 = c0_v[ti, :]
            g1 = c1_v[ti, :]
            for v in range(nv):
                sl = pl.ds(v * 16, 16)
                buf0_v[t, sl] = g0 * buf0_v[t, sl] + g1 * buf1_v[t, sl]
            return _

        lax.fori_loop(0, half, body, None)
        pltpu.sync_copy(buf0_v,
                        out_hbm.at[pl.ds(wid * tpw + r * half, half)])


def _mlp_kernel(xe_ref, valid_ref, w1_ref, b1_ref, w2_ref, b2_ref, y_ref):
    vm = valid_ref[0, 0]
    xe = jnp.where(vm > 0, xe_ref[0, 0], 0.0)
    h = jnp.dot(xe, w1_ref[0], preferred_element_type=jnp.float32) + b1_ref[0]
    h = jax.nn.gelu(h)
    y = jnp.dot(h, w2_ref[0], preferred_element_type=jnp.float32) + b2_ref[0]
    y_ref[0, 0] = y


def kernel(inputs, w_router, w1, b1, w2, b2):
    b, s, d = inputs.shape
    G = (b * s) // GS
    N = G * GS
    MLP = w1.shape[2]
    x2 = inputs.reshape(N, d)

    dst, r0, r1, c0, c1, valid, aux = pl.pallas_call(
        functools.partial(_router_kernel, G=G),
        out_shape=[
            jax.ShapeDtypeStruct((NW, K * N // NW // 128, 128), jnp.int32),
            jax.ShapeDtypeStruct((NW, N // NW), jnp.int32),
            jax.ShapeDtypeStruct((NW, N // NW), jnp.int32),
            jax.ShapeDtypeStruct((NW, N // NW, 16), jnp.float32),
            jax.ShapeDtypeStruct((NW, N // NW, 16), jnp.float32),
            jax.ShapeDtypeStruct((G, E, CAP, 1), jnp.float32),
            jax.ShapeDtypeStruct((1, 1), jnp.float32),
        ],
        interpret=_INTERPRET,
    )(x2, w_router)

    mesh = plsc.VectorSubcoreMesh(core_axis_name="c", subcore_axis_name="s")

    xe_flat = pl.kernel(
        _dispatch_sc,
        mesh=mesh,
        out_type=jax.ShapeDtypeStruct((NSLOT + 8, d), jnp.float32),
        scratch_types=[
            pltpu.VMEM((2, 128), jnp.int32),
            pltpu.VMEM((128, d), jnp.float32),
            pltpu.SemaphoreType.DMA,
        ],
    )(x2, dst)

    xe = xe_flat[:NSLOT].reshape(E, G, CAP, d)
    validT = valid.transpose(1, 0, 2, 3)
    b1r = b1.reshape(E, 1, MLP)
    b2r = b2.reshape(E, 1, d)

    y = pl.pallas_call(
        _mlp_kernel,
        grid=(E, G),
        in_specs=[
            pl.BlockSpec((1, 1, CAP, d), lambda e, g: (e, g, 0, 0)),
            pl.BlockSpec((1, 1, CAP, 1), lambda e, g: (e, g, 0, 0)),
            pl.BlockSpec((1, d, MLP), lambda e, g: (e, 0, 0)),
            pl.BlockSpec((1, 1, MLP), lambda e, g: (e, 0, 0)),
            pl.BlockSpec((1, MLP, d), lambda e, g: (e, 0, 0)),
            pl.BlockSpec((1, 1, d), lambda e, g: (e, 0, 0)),
        ],
        out_specs=pl.BlockSpec((1, 1, CAP, d), lambda e, g: (e, g, 0, 0)),
        out_shape=jax.ShapeDtypeStruct((E, G, CAP, d), jnp.float32),
        interpret=_INTERPRET,
    )(xe, validT, w1, b1r, w2, b2r)

    y_flat = y.reshape(E * G * CAP, d)
    out2 = pl.kernel(
        _combine_sc,
        mesh=mesh,
        out_type=jax.ShapeDtypeStruct((N, d), jnp.float32),
        scratch_types=[
            pltpu.VMEM((128,), jnp.int32),
            pltpu.VMEM((128,), jnp.int32),
            pltpu.VMEM((128, 16), jnp.float32),
            pltpu.VMEM((128, 16), jnp.float32),
            pltpu.VMEM((32, d), jnp.float32),
            pltpu.VMEM((32, d), jnp.float32),
            pltpu.SemaphoreType.DMA,
        ],
    )(y_flat, r0, r1, c0, c1)

    out = out2.reshape(b, s, d)
    aux_s = aux[0, 0]
    return out, {"auxiliary_loss": aux_s, "importance_loss": aux_s}


# speedup vs baseline: 1.1091x; 1.0503x over previous
import functools

import jax
import jax.numpy as jnp
from jax import lax
from jax.experimental import pallas as pl
from jax.experimental.pallas import tpu as pltpu
from jax.experimental.pallas import tpu_sc as plsc

_INTERPRET = False

GS = 1024
E = 8
K = 2
CAP = 256
NW = 32


def _router_kernel(x_ref, wr_ref,
                   topi_ref, pos_ref, keep_ref,
                   r0_ref, r1_ref, c0_ref, c1_ref, aux_ref,
                   *, G):
    x = x_ref[...]
    wr = wr_ref[...]
    N = G * GS
    logits = jnp.dot(x, wr, preferred_element_type=jnp.float32)
    m = jnp.max(logits, axis=-1, keepdims=True)
    ex = jnp.exp(logits - m)
    gates = ex / jnp.sum(ex, axis=-1, keepdims=True)

    idx8 = lax.broadcasted_iota(jnp.int32, (N, E), 1)
    top1v = jnp.max(gates, axis=-1, keepdims=True)
    top1i = jnp.min(jnp.where(gates == top1v, idx8, E), axis=-1, keepdims=True)
    oh1 = idx8 == top1i
    masked = jnp.where(oh1, -1.0, gates)
    top2v = jnp.max(masked, axis=-1, keepdims=True)
    top2i = jnp.min(jnp.where(masked == top2v, idx8, E), axis=-1, keepdims=True)
    oh2 = idx8 == top2i

    M = jnp.concatenate([oh1.astype(jnp.float32).reshape(G, GS, E),
                         oh2.astype(jnp.float32).reshape(G, GS, E)], axis=1)
    C = M
    sh = 1
    while sh < K * GS:
        C = C + jnp.concatenate(
            [jnp.zeros((G, sh, E), jnp.float32), C[:, :-sh, :]], axis=1)
        sh *= 2
    P = C - 1.0
    pos_entry = jnp.sum(M * P, axis=-1)
    keepf = (pos_entry < CAP).astype(jnp.float32)
    keepb = pos_entry < CAP
    pos_i = jnp.minimum(pos_entry, CAP - 1).astype(jnp.int32)

    topi_flat = jnp.concatenate([top1i.reshape(G, GS), top2i.reshape(G, GS)],
                                axis=1)
    topi_ref[...] = topi_flat.reshape(G, 1, K * GS)
    pos_ref[...] = pos_i.reshape(G, 1, K * GS)
    keep_ref[...] = keepf.reshape(G, 1, K * GS)

    gi = lax.broadcasted_iota(jnp.int32, (G, K * GS), 0)
    slot = (topi_flat * G + gi) * CAP + pos_i
    r0 = jnp.where(keepb[:, :GS], slot[:, :GS], 0).reshape(N)
    r1 = jnp.where(keepb[:, GS:], slot[:, GS:], 0).reshape(N)
    r0_ref[...] = r0.reshape(NW, N // NW)
    r1_ref[...] = r1.reshape(NW, N // NW)
    cv0 = (top1v.reshape(G, GS) * keepf[:, :GS]).reshape(N, 1)
    cv1 = (top2v.reshape(G, GS) * keepf[:, GS:]).reshape(N, 1)
    c0_ref[...] = jnp.broadcast_to(cv0, (N, 16)).reshape(NW, N // NW, 16)
    c1_ref[...] = jnp.broadcast_to(cv1, (N, 16)).reshape(NW, N // NW, 16)

    imp = jnp.sum(gates.reshape(G, GS, E), axis=1)
    mu = jnp.mean(imp, axis=-1, keepdims=True)
    var = jnp.mean((imp - mu) ** 2, axis=-1, keepdims=True)
    aux_ref[...] = jnp.mean(var / (mu + 1e-10) ** 2).reshape(1, 1)


def _mlp_kernel(x_ref, w1_ref, b1_ref, w2_ref, b2_ref,
                topi_ref, pos_ref, keep_ref, y_ref):
    e = pl.program_id(0)
    t = topi_ref[0]
    p = pos_ref[0]
    kp = keep_ref[0]
    t0, t1 = t[:, :GS], t[:, GS:]
    p0, p1 = p[:, :GS], p[:, GS:]
    k0, k1 = kp[:, :GS], kp[:, GS:]

    ci = lax.broadcasted_iota(jnp.int32, (CAP, GS), 0)
    oh0 = ((p0 == ci) & (t0 == e) & (k0 > 0)).astype(jnp.float32)
    oh1 = ((p1 == ci) & (t1 == e) & (k1 > 0)).astype(jnp.float32)
    dispT = oh0 + oh1

    xe = jnp.dot(dispT, x_ref[0], preferred_element_type=jnp.float32)
    h = jnp.dot(xe, w1_ref[0], preferred_element_type=jnp.float32) + b1_ref[0]
    h = jax.nn.gelu(h)
    y = jnp.dot(h, w2_ref[0], preferred_element_type=jnp.float32) + b2_ref[0]
    y_ref[0, 0] = y


def _combine_sc(y_hbm, r0_hbm, r1_hbm, c0_hbm, c1_hbm, out_hbm,
                idx0_v, idx1_v, c0_v, c1_v, buf0_v, buf1_v, sem):
    wid = lax.axis_index("s") * 2 + lax.axis_index("c")
    tpw = 128
    half = 32
    d = buf0_v.shape[1]
    nv = d // 16
    pltpu.sync_copy(r0_hbm.at[wid], idx0_v)
    pltpu.sync_copy(r1_hbm.at[wid], idx1_v)
    pltpu.sync_copy(c0_hbm.at[wid], c0_v)
    pltpu.sync_copy(c1_hbm.at[wid], c1_v)
    for r in range(4):
        cp0 = pltpu.async_copy(y_hbm.at[idx0_v.at[pl.ds(r * half, half)]],
                               buf0_v, sem)
        cp1 = pltpu.async_copy(y_hbm.at[idx1_v.at[pl.ds(r * half, half)]],
                               buf1_v, sem)
        cp0.wait()
        cp1.wait()

        def body(t, _, r=r):
            ti = r * half + t
            g0 = c0_v[ti, :]
            g1 = c1_v[ti, :]
            for v in range(nv):
                sl = pl.ds(v * 16, 16)
                buf0_v[t, sl] = g0 * buf0_v[t, sl] + g1 * buf1_v[t, sl]
            return _

        lax.fori_loop(0, half, body, None)
        pltpu.sync_copy(buf0_v,
                        out_hbm.at[pl.ds(wid * tpw + r * half, half)])


def kernel(inputs, w_router, w1, b1, w2, b2):
    b, s, d = inputs.shape
    G = (b * s) // GS
    N = G * GS
    MLP = w1.shape[2]
    x2 = inputs.reshape(N, d)

    topi, pos, keep, r0, r1, c0, c1, aux = pl.pallas_call(
        functools.partial(_router_kernel, G=G),
        out_shape=[
            jax.ShapeDtypeStruct((G, 1, K * GS), jnp.int32),
            jax.ShapeDtypeStruct((G, 1, K * GS), jnp.int32),
            jax.ShapeDtypeStruct((G, 1, K * GS), jnp.float32),
            jax.ShapeDtypeStruct((NW, N // NW), jnp.int32),
            jax.ShapeDtypeStruct((NW, N // NW), jnp.int32),
            jax.ShapeDtypeStruct((NW, N // NW, 16), jnp.float32),
            jax.ShapeDtypeStruct((NW, N // NW, 16), jnp.float32),
            jax.ShapeDtypeStruct((1, 1), jnp.float32),
        ],
        interpret=_INTERPRET,
    )(x2, w_router)

    x3 = inputs.reshape(G, GS, d)
    b1r = b1.reshape(E, 1, MLP)
    b2r = b2.reshape(E, 1, d)

    y = pl.pallas_call(
        _mlp_kernel,
        grid=(E, G),
        in_specs=[
            pl.BlockSpec((1, GS, d), lambda e, g: (g, 0, 0)),
            pl.BlockSpec((1, d, MLP), lambda e, g: (e, 0, 0)),
            pl.BlockSpec((1, 1, MLP), lambda e, g: (e, 0, 0)),
            pl.BlockSpec((1, MLP, d), lambda e, g: (e, 0, 0)),
            pl.BlockSpec((1, 1, d), lambda e, g: (e, 0, 0)),
            pl.BlockSpec((1, 1, K * GS), lambda e, g: (g, 0, 0)),
            pl.BlockSpec((1, 1, K * GS), lambda e, g: (g, 0, 0)),
            pl.BlockSpec((1, 1, K * GS), lambda e, g: (g, 0, 0)),
        ],
        out_specs=pl.BlockSpec((1, 1, CAP, d), lambda e, g: (e, g, 0, 0)),
        out_shape=jax.ShapeDtypeStruct((E, G, CAP, d), jnp.float32),
        interpret=_INTERPRET,
    )(x3, w1, b1r, w2, b2r, topi, pos, keep)

    mesh = plsc.VectorSubcoreMesh(core_axis_name="c", subcore_axis_name="s")
    y_flat = y.reshape(E * G * CAP, d)
    out2 = pl.kernel(
        _combine_sc,
        mesh=mesh,
        out_type=jax.ShapeDtypeStruct((N, d), jnp.float32),
        scratch_types=[
            pltpu.VMEM((128,), jnp.int32),
            pltpu.VMEM((128,), jnp.int32),
            pltpu.VMEM((128, 16), jnp.float32),
            pltpu.VMEM((128, 16), jnp.float32),
            pltpu.VMEM((32, d), jnp.float32),
            pltpu.VMEM((32, d), jnp.float32),
            pltpu.SemaphoreType.DMA,
        ],
    )(y_flat, r0, r1, c0, c1)

    out = out2.reshape(b, s, d)
    aux_s = aux[0, 0]
    return out, {"auxiliary_loss": aux_s, "importance_loss": aux_s}


# speedup vs baseline: 1.2425x; 1.1203x over previous
import functools

import jax
import jax.numpy as jnp
from jax import lax
from jax.experimental import pallas as pl
from jax.experimental.pallas import tpu as pltpu
from jax.experimental.pallas import tpu_sc as plsc

_INTERPRET = False

GS = 1024
E = 8
K = 2
CAP = 256
NW = 32


def _router_kernel(x_ref, wr_ref,
                   topi_ref, pos_ref, keep_ref,
                   r0_ref, r1_ref, c0_ref, c1_ref, aux_ref,
                   *, G):
    x = x_ref[...]
    wr = wr_ref[...]
    N = G * GS
    logits = jnp.dot(x, wr, preferred_element_type=jnp.float32)
    m = jnp.max(logits, axis=-1, keepdims=True)
    ex = jnp.exp(logits - m)
    gates = ex / jnp.sum(ex, axis=-1, keepdims=True)

    idx8 = lax.broadcasted_iota(jnp.int32, (N, E), 1)
    top1v = jnp.max(gates, axis=-1, keepdims=True)
    top1i = jnp.min(jnp.where(gates == top1v, idx8, E), axis=-1, keepdims=True)
    oh1 = idx8 == top1i
    masked = jnp.where(oh1, -1.0, gates)
    top2v = jnp.max(masked, axis=-1, keepdims=True)
    top2i = jnp.min(jnp.where(masked == top2v, idx8, E), axis=-1, keepdims=True)
    oh2 = idx8 == top2i

    M = jnp.concatenate([oh1.astype(jnp.float32).reshape(G, GS, E),
                         oh2.astype(jnp.float32).reshape(G, GS, E)], axis=1)
    C = M
    sh = 1
    while sh < K * GS:
        C = C + jnp.concatenate(
            [jnp.zeros((G, sh, E), jnp.float32), C[:, :-sh, :]], axis=1)
        sh *= 2
    P = C - 1.0
    pos_entry = jnp.sum(M * P, axis=-1)
    keepf = (pos_entry < CAP).astype(jnp.float32)
    keepb = pos_entry < CAP
    pos_i = jnp.minimum(pos_entry, CAP - 1).astype(jnp.int32)

    topi_flat = jnp.concatenate([top1i.reshape(G, GS), top2i.reshape(G, GS)],
                                axis=1)
    topi_ref[...] = topi_flat.reshape(G, 1, K * GS)
    pos_ref[...] = pos_i.reshape(G, 1, K * GS)
    keep_ref[...] = keepf.reshape(G, 1, K * GS)

    gi = lax.broadcasted_iota(jnp.int32, (G, K * GS), 0)
    slot = (topi_flat * G + gi) * CAP + pos_i
    r0 = jnp.where(keepb[:, :GS], slot[:, :GS], 0).reshape(N)
    r1 = jnp.where(keepb[:, GS:], slot[:, GS:], 0).reshape(N)
    r0_ref[...] = r0.reshape(NW, N // NW)
    r1_ref[...] = r1.reshape(NW, N // NW)
    cv0 = (top1v.reshape(G, GS) * keepf[:, :GS]).reshape(N, 1)
    cv1 = (top2v.reshape(G, GS) * keepf[:, GS:]).reshape(N, 1)
    c0_ref[...] = jnp.broadcast_to(cv0, (N, 16)).reshape(NW, N // NW, 16)
    c1_ref[...] = jnp.broadcast_to(cv1, (N, 16)).reshape(NW, N // NW, 16)

    imp = jnp.sum(gates.reshape(G, GS, E), axis=1)
    mu = jnp.mean(imp, axis=-1, keepdims=True)
    var = jnp.mean((imp - mu) ** 2, axis=-1, keepdims=True)
    aux_ref[...] = jnp.mean(var / (mu + 1e-10) ** 2).reshape(1, 1)


def _mlp_kernel(x_ref, w1_ref, b1_ref, w2_ref, b2_ref,
                topi_ref, pos_ref, keep_ref, y_ref):
    e = pl.program_id(0)
    ci = lax.broadcasted_iota(jnp.int32, (CAP, GS), 0)
    xes = []
    for q in range(2):
        t = topi_ref[q]
        p = pos_ref[q]
        kp = keep_ref[q]
        t0, t1 = t[:, :GS], t[:, GS:]
        p0, p1 = p[:, :GS], p[:, GS:]
        k0, k1 = kp[:, :GS], kp[:, GS:]
        oh0 = ((p0 == ci) & (t0 == e) & (k0 > 0)).astype(jnp.float32)
        oh1 = ((p1 == ci) & (t1 == e) & (k1 > 0)).astype(jnp.float32)
        dispT = oh0 + oh1
        xes.append(jnp.dot(dispT, x_ref[q], preferred_element_type=jnp.float32))
    xe = jnp.concatenate(xes, axis=0)
    h = jnp.dot(xe, w1_ref[0], preferred_element_type=jnp.float32) + b1_ref[0]
    h = jax.nn.gelu(h)
    y = jnp.dot(h, w2_ref[0], preferred_element_type=jnp.float32) + b2_ref[0]
    y_ref[0, 0] = y[:CAP]
    y_ref[0, 1] = y[CAP:]


def _combine_sc(y_hbm, r0_hbm, r1_hbm, c0_hbm, c1_hbm, out_hbm,
                idx0_v, idx1_v, c0_v, c1_v, buf0_v, buf1_v, sem):
    wid = lax.axis_index("s") * 2 + lax.axis_index("c")
    tpw = 128
    half = 32
    d = buf0_v.shape[1]
    nv = d // 16
    pltpu.sync_copy(r0_hbm.at[wid], idx0_v)
    pltpu.sync_copy(r1_hbm.at[wid], idx1_v)
    pltpu.sync_copy(c0_hbm.at[wid], c0_v)
    pltpu.sync_copy(c1_hbm.at[wid], c1_v)
    for r in range(4):
        cp0 = pltpu.async_copy(y_hbm.at[idx0_v.at[pl.ds(r * half, half)]],
                               buf0_v, sem)
        cp1 = pltpu.async_copy(y_hbm.at[idx1_v.at[pl.ds(r * half, half)]],
                               buf1_v, sem)
        cp0.wait()
        cp1.wait()

        def body(t, _, r=r):
            ti = r * half + t
            g0 = c0_v[ti, :]
            g1 = c1_v[ti, :]
            for v in range(nv):
                sl = pl.ds(v * 16, 16)
                buf0_v[t, sl] = g0 * buf0_v[t, sl] + g1 * buf1_v[t, sl]
            return _

        lax.fori_loop(0, half, body, None)
        pltpu.sync_copy(buf0_v,
                        out_hbm.at[pl.ds(wid * tpw + r * half, half)])


def kernel(inputs, w_router, w1, b1, w2, b2):
    b, s, d = inputs.shape
    G = (b * s) // GS
    N = G * GS
    MLP = w1.shape[2]
    x2 = inputs.reshape(N, d)

    topi, pos, keep, r0, r1, c0, c1, aux = pl.pallas_call(
        functools.partial(_router_kernel, G=G),
        out_shape=[
            jax.ShapeDtypeStruct((G, 1, K * GS), jnp.int32),
            jax.ShapeDtypeStruct((G, 1, K * GS), jnp.int32),
            jax.ShapeDtypeStruct((G, 1, K * GS), jnp.float32),
            jax.ShapeDtypeStruct((NW, N // NW), jnp.int32),
            jax.ShapeDtypeStruct((NW, N // NW), jnp.int32),
            jax.ShapeDtypeStruct((NW, N // NW, 16), jnp.float32),
            jax.ShapeDtypeStruct((NW, N // NW, 16), jnp.float32),
            jax.ShapeDtypeStruct((1, 1), jnp.float32),
        ],
        interpret=_INTERPRET,
    )(x2, w_router)

    x3 = inputs.reshape(G, GS, d)
    b1r = b1.reshape(E, 1, MLP)
    b2r = b2.reshape(E, 1, d)

    y = pl.pallas_call(
        _mlp_kernel,
        grid=(E, G // 2),
        in_specs=[
            pl.BlockSpec((2, GS, d), lambda e, g: (g, 0, 0)),
            pl.BlockSpec((1, d, MLP), lambda e, g: (e, 0, 0)),
            pl.BlockSpec((1, 1, MLP), lambda e, g: (e, 0, 0)),
            pl.BlockSpec((1, MLP, d), lambda e, g: (e, 0, 0)),
            pl.BlockSpec((1, 1, d), lambda e, g: (e, 0, 0)),
            pl.BlockSpec((2, 1, K * GS), lambda e, g: (g, 0, 0)),
            pl.BlockSpec((2, 1, K * GS), lambda e, g: (g, 0, 0)),
            pl.BlockSpec((2, 1, K * GS), lambda e, g: (g, 0, 0)),
        ],
        out_specs=pl.BlockSpec((1, 2, CAP, d), lambda e, g: (e, g, 0, 0)),
        out_shape=jax.ShapeDtypeStruct((E, G, CAP, d), jnp.float32),
        interpret=_INTERPRET,
    )(x3, w1, b1r, w2, b2r, topi, pos, keep)

    mesh = plsc.VectorSubcoreMesh(core_axis_name="c", subcore_axis_name="s")
    y_flat = y.reshape(E * G * CAP, d)
    out2 = pl.kernel(
        _combine_sc,
        mesh=mesh,
        out_type=jax.ShapeDtypeStruct((N, d), jnp.float32),
        scratch_types=[
            pltpu.VMEM((128,), jnp.int32),
            pltpu.VMEM((128,), jnp.int32),
            pltpu.VMEM((128, 16), jnp.float32),
            pltpu.VMEM((128, 16), jnp.float32),
            pltpu.VMEM((32, d), jnp.float32),
            pltpu.VMEM((32, d), jnp.float32),
            pltpu.SemaphoreType.DMA,
        ],
    )(y_flat, r0, r1, c0, c1)

    out = out2.reshape(b, s, d)
    aux_s = aux[0, 0]
    return out, {"auxiliary_loss": aux_s, "importance_loss": aux_s}


# speedup vs baseline: 1.2574x; 1.0120x over previous
import functools

import jax
import jax.numpy as jnp
from jax import lax
from jax.experimental import pallas as pl
from jax.experimental.pallas import tpu as pltpu
from jax.experimental.pallas import tpu_sc as plsc

_INTERPRET = False

GS = 1024
E = 8
K = 2
CAP = 256
NW = 32


def _router_kernel(x_ref, wr_ref,
                   topi_ref, pos_ref, keep_ref,
                   r0_ref, r1_ref, c0_ref, c1_ref, aux_ref,
                   *, G):
    x = x_ref[...]
    wr = wr_ref[...]
    N = G * GS
    logits = jnp.dot(x, wr, preferred_element_type=jnp.float32)
    m = jnp.max(logits, axis=-1, keepdims=True)
    ex = jnp.exp(logits - m)
    gates = ex / jnp.sum(ex, axis=-1, keepdims=True)

    idx8 = lax.broadcasted_iota(jnp.int32, (N, E), 1)
    top1v = jnp.max(gates, axis=-1, keepdims=True)
    top1i = jnp.min(jnp.where(gates == top1v, idx8, E), axis=-1, keepdims=True)
    oh1 = idx8 == top1i
    masked = jnp.where(oh1, -1.0, gates)
    top2v = jnp.max(masked, axis=-1, keepdims=True)
    top2i = jnp.min(jnp.where(masked == top2v, idx8, E), axis=-1, keepdims=True)
    oh2 = idx8 == top2i

    M = jnp.concatenate([oh1.astype(jnp.float32).reshape(G, GS, E),
                         oh2.astype(jnp.float32).reshape(G, GS, E)], axis=1)
    C = M
    sh = 1
    while sh < K * GS:
        C = C + jnp.concatenate(
            [jnp.zeros((G, sh, E), jnp.float32), C[:, :-sh, :]], axis=1)
        sh *= 2
    P = C - 1.0
    pos_entry = jnp.sum(M * P, axis=-1)
    keepf = (pos_entry < CAP).astype(jnp.float32)
    keepb = pos_entry < CAP
    pos_i = jnp.minimum(pos_entry, CAP - 1).astype(jnp.int32)

    topi_flat = jnp.concatenate([top1i.reshape(G, GS), top2i.reshape(G, GS)],
                                axis=1)
    topi_ref[...] = topi_flat.reshape(G, 1, K * GS)
    pos_ref[...] = pos_i.reshape(G, 1, K * GS)
    keep_ref[...] = keepf.reshape(G, 1, K * GS)

    gi = lax.broadcasted_iota(jnp.int32, (G, K * GS), 0)
    slot = (topi_flat * G + gi) * CAP + pos_i
    r0 = jnp.where(keepb[:, :GS], slot[:, :GS], 0).reshape(N)
    r1 = jnp.where(keepb[:, GS:], slot[:, GS:], 0).reshape(N)
    r0_ref[...] = r0.reshape(NW, N // NW)
    r1_ref[...] = r1.reshape(NW, N // NW)
    cv0 = (top1v.reshape(G, GS) * keepf[:, :GS]).reshape(N, 1)
    cv1 = (top2v.reshape(G, GS) * keepf[:, GS:]).reshape(N, 1)
    c0_ref[...] = jnp.broadcast_to(cv0, (N, 16)).reshape(NW, N // NW, 16)
    c1_ref[...] = jnp.broadcast_to(cv1, (N, 16)).reshape(NW, N // NW, 16)

    imp = jnp.sum(gates.reshape(G, GS, E), axis=1)
    mu = jnp.mean(imp, axis=-1, keepdims=True)
    var = jnp.mean((imp - mu) ** 2, axis=-1, keepdims=True)
    aux_ref[...] = jnp.mean(var / (mu + 1e-10) ** 2).reshape(1, 1)


def _mlp_kernel(x_ref, w1_ref, b1_ref, w2_ref, b2_ref,
                topi_ref, pos_ref, keep_ref, y_ref):
    e = pl.program_id(0)
    ci = lax.broadcasted_iota(jnp.int32, (CAP, GS), 0)
    xes = []
    for q in range(2):
        t = topi_ref[q]
        p = pos_ref[q]
        kp = keep_ref[q]
        t0, t1 = t[:, :GS], t[:, GS:]
        p0, p1 = p[:, :GS], p[:, GS:]
        k0, k1 = kp[:, :GS], kp[:, GS:]
        oh0 = ((p0 == ci) & (t0 == e) & (k0 > 0)).astype(jnp.float32)
        oh1 = ((p1 == ci) & (t1 == e) & (k1 > 0)).astype(jnp.float32)
        dispT = oh0 + oh1
        xes.append(jnp.dot(dispT, x_ref[q], preferred_element_type=jnp.float32))
    xe = jnp.concatenate(xes, axis=0)
    h = jnp.dot(xe, w1_ref[0], preferred_element_type=jnp.float32) + b1_ref[0]
    h = jax.nn.gelu(h)
    y = jnp.dot(h, w2_ref[0], preferred_element_type=jnp.float32) + b2_ref[0]
    y_ref[0, 0] = y[:CAP]
    y_ref[0, 1] = y[CAP:]


def _combine_sc(y_hbm, r0_hbm, r1_hbm, c0_hbm, c1_hbm, out_hbm,
                idx0_v, idx1_v, c0_v, c1_v,
                a0_v, a1_v, b0_v, b1_v, sem_a, sem_b, sem_o):
    wid = lax.axis_index("s") * 2 + lax.axis_index("c")
    tpw = 128
    step = 16
    R = tpw // step
    d = a0_v.shape[1]
    nv = d // 16
    pltpu.sync_copy(r0_hbm.at[wid], idx0_v)
    pltpu.sync_copy(r1_hbm.at[wid], idx1_v)
    pltpu.sync_copy(c0_hbm.at[wid], c0_v)
    pltpu.sync_copy(c1_hbm.at[wid], c1_v)
    bufs = [(a0_v, a1_v), (b0_v, b1_v)]
    sems = [sem_a, sem_b]

    def gather(r, pair, sem):
        cg0 = pltpu.async_copy(y_hbm.at[idx0_v.at[pl.ds(r * step, step)]],
                               pair[0], sem)
        cg1 = pltpu.async_copy(y_hbm.at[idx1_v.at[pl.ds(r * step, step)]],
                               pair[1], sem)
        return cg0, cg1

    g_in = gather(0, bufs[0], sems[0])
    writes = []
    for r in range(R):
        cur0, cur1 = bufs[r % 2]
        g_in[0].wait()
        g_in[1].wait()
        if r < R - 1:
            if r >= 1:
                writes[r - 1].wait()
            g_in = gather(r + 1, bufs[(r + 1) % 2], sems[(r + 1) % 2])

        def body(t, _, r=r):
            ti = r * step + t
            g0 = c0_v[ti, :]
            g1 = c1_v[ti, :]
            for v in range(nv):
                sl = pl.ds(v * 16, 16)
                cur0[t, sl] = g0 * cur0[t, sl] + g1 * cur1[t, sl]
            return _

        lax.fori_loop(0, step, body, None)
        writes.append(pltpu.async_copy(
            cur0, out_hbm.at[pl.ds(wid * tpw + r * step, step)], sem_o))
    writes[R - 2].wait()
    writes[R - 1].wait()


def kernel(inputs, w_router, w1, b1, w2, b2):
    b, s, d = inputs.shape
    G = (b * s) // GS
    N = G * GS
    MLP = w1.shape[2]
    x2 = inputs.reshape(N, d)

    topi, pos, keep, r0, r1, c0, c1, aux = pl.pallas_call(
        functools.partial(_router_kernel, G=G),
        out_shape=[
            jax.ShapeDtypeStruct((G, 1, K * GS), jnp.int32),
            jax.ShapeDtypeStruct((G, 1, K * GS), jnp.int32),
            jax.ShapeDtypeStruct((G, 1, K * GS), jnp.float32),
            jax.ShapeDtypeStruct((NW, N // NW), jnp.int32),
            jax.ShapeDtypeStruct((NW, N // NW), jnp.int32),
            jax.ShapeDtypeStruct((NW, N // NW, 16), jnp.float32),
            jax.ShapeDtypeStruct((NW, N // NW, 16), jnp.float32),
            jax.ShapeDtypeStruct((1, 1), jnp.float32),
        ],
        interpret=_INTERPRET,
    )(x2, w_router)

    x3 = inputs.reshape(G, GS, d)
    b1r = b1.reshape(E, 1, MLP)
    b2r = b2.reshape(E, 1, d)

    y = pl.pallas_call(
        _mlp_kernel,
        grid=(E, G // 2),
        in_specs=[
            pl.BlockSpec((2, GS, d), lambda e, g: (g, 0, 0)),
            pl.BlockSpec((1, d, MLP), lambda e, g: (e, 0, 0)),
            pl.BlockSpec((1, 1, MLP), lambda e, g: (e, 0, 0)),
            pl.BlockSpec((1, MLP, d), lambda e, g: (e, 0, 0)),
            pl.BlockSpec((1, 1, d), lambda e, g: (e, 0, 0)),
            pl.BlockSpec((2, 1, K * GS), lambda e, g: (g, 0, 0)),
            pl.BlockSpec((2, 1, K * GS), lambda e, g: (g, 0, 0)),
            pl.BlockSpec((2, 1, K * GS), lambda e, g: (g, 0, 0)),
        ],
        out_specs=pl.BlockSpec((1, 2, CAP, d), lambda e, g: (e, g, 0, 0)),
        out_shape=jax.ShapeDtypeStruct((E, G, CAP, d), jnp.float32),
        interpret=_INTERPRET,
    )(x3, w1, b1r, w2, b2r, topi, pos, keep)

    mesh = plsc.VectorSubcoreMesh(core_axis_name="c", subcore_axis_name="s")
    y_flat = y.reshape(E * G * CAP, d)
    out2 = pl.kernel(
        _combine_sc,
        mesh=mesh,
        out_type=jax.ShapeDtypeStruct((N, d), jnp.float32),
        scratch_types=[
            pltpu.VMEM((128,), jnp.int32),
            pltpu.VMEM((128,), jnp.int32),
            pltpu.VMEM((128, 16), jnp.float32),
            pltpu.VMEM((128, 16), jnp.float32),
            pltpu.VMEM((16, d), jnp.float32),
            pltpu.VMEM((16, d), jnp.float32),
            pltpu.VMEM((16, d), jnp.float32),
            pltpu.VMEM((16, d), jnp.float32),
            pltpu.SemaphoreType.DMA,
            pltpu.SemaphoreType.DMA,
            pltpu.SemaphoreType.DMA,
        ],
    )(y_flat, r0, r1, c0, c1)

    out = out2.reshape(b, s, d)
    aux_s = aux[0, 0]
    return out, {"auxiliary_loss": aux_s, "importance_loss": aux_s}
